# Initial kernel scaffold; baseline (speedup 1.0000x reference)
import jax
import jax.numpy as jnp
from jax.experimental import pallas as pl


def kernel(x_feat, intrins, depth_x, params):
    raise NotImplementedError("write your pallas kernel here")


# speedup vs baseline: 1.2634x; 1.2634x over previous
import jax
import jax.numpy as jnp
from jax.experimental import pallas as pl

B = 8
IN_C = 256
OUT_C = 80
D = 41
FH = 24
FW = 24
IH = 384
IW = 384
NX, NY, NZ = 200, 200, 1


def _conv(x, w, b, stride, pad):
    o = jax.lax.conv_general_dilated(x, w, (stride, stride), ((pad, pad), (pad, pad)), dimension_numbers=('NCHW', 'OIHW', 'NCHW'))
    return o + b[None, :, None, None]


def _bn(x, g, be):
    m = x.mean(axis=(0, 2, 3), keepdims=True)
    v = x.var(axis=(0, 2, 3), keepdims=True)
    return g[None, :, None, None] * (x - m) / jnp.sqrt(v + 1e-5) + be[None, :, None, None]


def _frustum():
    ds = jnp.arange(4.0, 45.0, 1.0).reshape(-1, 1, 1) * jnp.ones((1, FH, FW), jnp.float32)
    xs = jnp.linspace(0.0, IW - 1.0, FW).reshape(1, 1, FW) * jnp.ones((D, FH, 1), jnp.float32)
    ys = jnp.linspace(0.0, IH - 1.0, FH).reshape(1, FH, 1) * jnp.ones((D, 1, FW), jnp.float32)
    return jnp.stack([xs, ys, ds], -1)


def _dtransform(d, p):
    x = jax.nn.relu(_bn(_conv(d, p['dt_w1'], p['dt_b1'], 1, 0), p['dt_g1'], p['dt_be1']))
    x = jax.nn.relu(_bn(_conv(x, p['dt_w2'], p['dt_b2'], 4, 2), p['dt_g2'], p['dt_be2']))
    x = jax.nn.relu(_bn(_conv(x, p['dt_w3'], p['dt_b3'], 2, 2), p['dt_g3'], p['dt_be3']))
    x = jax.nn.relu(_bn(_conv(x, p['dt_w4'], p['dt_b4'], 2, 2), p['dt_g4'], p['dt_be4']))
    return x


def _depthnet(x, p):
    x = jax.nn.relu(_bn(_conv(x, p['dn_w1'], p['dn_b1'], 1, 1), p['dn_g1'], p['dn_be1']))
    x = jax.nn.relu(_bn(_conv(x, p['dn_w2'], p['dn_b2'], 1, 1), p['dn_g2'], p['dn_be2']))
    x = _conv(x, p['dn_w3'], p['dn_b3'], 1, 0)
    return x


def _crop_kernel(inp_ref, out_ref):
    out_ref[...] = inp_ref[:, :, :, 50:]


def kernel(x_feat, intrins, depth_x, params):
    frustum = _frustum()
    pts = jnp.broadcast_to(frustum[None, None, ..., None], (B, 1, D, FH, FW, 3, 1))
    pts = jnp.concatenate([pts[..., :2, :] * pts[..., 2:3, :], pts[..., 2:3, :]], axis=5)
    rots = jnp.array([[1.0, 0.0, 0.0], [0.0, 0.0, 1.0], [0.0, -1.0, 0.0]], jnp.float32).reshape(1, 1, 3, 3)
    combine = jnp.matmul(rots, jnp.linalg.inv(intrins))
    geom = jnp.matmul(combine.reshape(B, 1, 1, 1, 1, 3, 3), pts)[..., 0]
    x = x_feat.reshape(B, IN_C, FH, FW)
    _d = _dtransform(depth_x, params)
    x = jnp.concatenate([_d, x], axis=1)
    x = _depthnet(x, params)
    avgd = depth_x.reshape(B, 1, FH, 16, FW, 16).mean(axis=(3, 5))
    mean_d = jnp.floor(avgd / 1000.0 * D)
    xg = jnp.broadcast_to(jnp.arange(D, dtype=jnp.float32).reshape(1, D, 1, 1), (B, D, FH, FW))
    sigma = 0.5
    gauss = (1.0 / jnp.sqrt(2.0 * jnp.pi)) * sigma * jnp.exp(-(xg - mean_d) ** 2 / 2.0 * sigma ** 2)
    depth = jax.nn.softmax(gauss, axis=1)
    x = depth[:, None, :, :, :] * x[:, :, None, :, :]
    x = x.reshape(B, 1, OUT_C, D, FH, FW).transpose(0, 1, 3, 4, 5, 2)
    dxv = jnp.array([0.5, 0.5, 20.0], jnp.float32)
    bxv = jnp.array([-49.75, -49.75, 0.0], jnp.float32)
    Np = B * 1 * D * FH * FW
    xf = x.reshape(Np, OUT_C)
    gf = ((geom - (bxv - dxv / 2.0)) / dxv).astype(jnp.int32).reshape(Np, 3)
    batch_ix = jnp.repeat(jnp.arange(B, dtype=jnp.int32), Np // B)
    kept = (gf[:, 0] >= 0) & (gf[:, 0] < NX) & (gf[:, 1] >= 0) & (gf[:, 1] < NY) & (gf[:, 2] >= 0) & (gf[:, 2] < NZ)
    xf = jnp.where(kept[:, None], xf, 0.0)
    gx = jnp.clip(gf[:, 0], 0, NX - 1)
    gy = jnp.clip(gf[:, 1], 0, NY - 1)
    gz = jnp.clip(gf[:, 2], 0, NZ - 1)
    flat = ((batch_ix * NZ + gz) * NX + gx) * NY + gy
    out = jnp.zeros((B * NZ * NX * NY, OUT_C), jnp.float32).at[flat].add(xf)
    out = out.reshape(B, NZ, NX, NY, OUT_C).transpose(0, 4, 1, 2, 3)
    full = jnp.concatenate([out[:, :, z] for z in range(NZ)], axis=1)
    cropped = pl.pallas_call(
        _crop_kernel,
        grid=(B, OUT_C * NZ // 8),
        in_specs=[pl.BlockSpec((1, 8, NX, NY), lambda b, c: (b, c, 0, 0))],
        out_specs=pl.BlockSpec((1, 8, NX, NY - 50), lambda b, c: (b, c, 0, 0)),
        out_shape=jax.ShapeDtypeStruct((B, OUT_C * NZ, NX, NY - 50), jnp.float32),
    )(full)
    return cropped


# speedup vs baseline: 1.2702x; 1.0054x over previous
import functools

import jax
import jax.numpy as jnp
from jax import lax
from jax.experimental import pallas as pl
from jax.experimental.pallas import tpu as pltpu
from jax.experimental.pallas import tpu_sc as plsc

B = 8
IN_C = 256
OUT_C = 80
D = 41
FH = 24
FW = 24
IH = 384
IW = 384
NX, NY, NZ = 200, 200, 1

HW = FH * FW
NPB = D * HW
CROP = 50
NYC = NY - CROP
XHALF = NX // 2
ROWS_HALF = XHALF * NYC
TRASH = 512
ACC_ROWS = ROWS_HALF + TRASH
ROWS_PER_SUB = 1000
HWT = HW // 16
DP = 48


def _conv(x, w, b, stride, pad):
    o = jax.lax.conv_general_dilated(x, w, (stride, stride), ((pad, pad), (pad, pad)), dimension_numbers=('NCHW', 'OIHW', 'NCHW'))
    return o + b[None, :, None, None]


def _bn(x, g, be):
    m = x.mean(axis=(0, 2, 3), keepdims=True)
    v = x.var(axis=(0, 2, 3), keepdims=True)
    return g[None, :, None, None] * (x - m) / jnp.sqrt(v + 1e-5) + be[None, :, None, None]


def _frustum():
    ds = jnp.arange(4.0, 45.0, 1.0).reshape(-1, 1, 1) * jnp.ones((1, FH, FW), jnp.float32)
    xs = jnp.linspace(0.0, IW - 1.0, FW).reshape(1, 1, FW) * jnp.ones((D, FH, 1), jnp.float32)
    ys = jnp.linspace(0.0, IH - 1.0, FH).reshape(1, FH, 1) * jnp.ones((D, 1, FW), jnp.float32)
    return jnp.stack([xs, ys, ds], -1)


def _dtransform(d, p):
    x = jax.nn.relu(_bn(_conv(d, p['dt_w1'], p['dt_b1'], 1, 0), p['dt_g1'], p['dt_be1']))
    x = jax.nn.relu(_bn(_conv(x, p['dt_w2'], p['dt_b2'], 4, 2), p['dt_g2'], p['dt_be2']))
    x = jax.nn.relu(_bn(_conv(x, p['dt_w3'], p['dt_b3'], 2, 2), p['dt_g3'], p['dt_be3']))
    x = jax.nn.relu(_bn(_conv(x, p['dt_w4'], p['dt_b4'], 2, 2), p['dt_g4'], p['dt_be4']))
    return x


def _depthnet(x, p):
    x = jax.nn.relu(_bn(_conv(x, p['dn_w1'], p['dn_b1'], 1, 1), p['dn_g1'], p['dn_be1']))
    x = jax.nn.relu(_bn(_conv(x, p['dn_w2'], p['dn_b2'], 1, 1), p['dn_g2'], p['dn_be2']))
    x = _conv(x, p['dn_w3'], p['dn_b3'], 1, 0)
    return x


def _splat_body(feat_hbm, wdep_hbm, gxf_hbm, gyf_hbm, gzf_hbm, zeros_hbm, out_hbm,
                acc, feat_v, pay_v, w_v, gx_v, gy_v, gz_v, idx_v):
    c = lax.axis_index("c")
    s = lax.axis_index("s")
    cx0 = c * XHALF
    lanes = lax.iota(jnp.int32, 16)

    def batch_body(b, carry):
        @pl.when(s == 0)
        def _zero():
            pltpu.sync_copy(zeros_hbm, acc.at[pl.ds(0, ROWS_HALF)])

        fbase = (b * HW + s * HWT) * OUT_C
        mbase = (b * HW + s * HWT) * DP
        pltpu.sync_copy(feat_hbm.at[pl.ds(fbase, HWT * OUT_C)], feat_v)
        pltpu.sync_copy(wdep_hbm.at[pl.ds(mbase, HWT * DP)], w_v)
        pltpu.sync_copy(gxf_hbm.at[pl.ds(mbase, HWT * DP)], gx_v)
        pltpu.sync_copy(gyf_hbm.at[pl.ds(mbase, HWT * DP)], gy_v)
        pltpu.sync_copy(gzf_hbm.at[pl.ds(mbase, HWT * DP)], gz_v)
        plsc.subcore_barrier()

        def pair_body(t, carry2):
            for u in range(2):
                hwl = 2 * t + u
                fk = [feat_v[pl.ds(hwl * OUT_C + k * 16, 16)] for k in range(5)]
                for g in range(3):
                    msl = pl.ds(hwl * DP + g * 16, 16)
                    fx = (gx_v[msl] + 50.0) / 0.5
                    fy = (gy_v[msl] + 50.0) / 0.5
                    fz = (gz_v[msl] + 10.0) / 20.0
                    okf = ((fx > -3e4) & (fx < 3e4) & (fy > -3e4) & (fy < 3e4)
                           & (fz > -3e4) & (fz < 3e4))
                    ix = jnp.where(okf, fx, -3e4).astype(jnp.int32)
                    iy = jnp.where(okf, fy, -3e4).astype(jnp.int32)
                    iz = jnp.where(okf, fz, -3e4).astype(jnp.int32)
                    keep = ((ix >= cx0) & (ix < cx0 + XHALF)
                            & (iy >= CROP) & (iy < NY) & (iz == 0))
                    widx = (ix - cx0) * NYC + (iy - CROP)
                    trash = ROWS_HALF + ((g * 16 + s * 89 + lanes) & (TRASH - 1))
                    idx_v[0, pl.ds(u * DP + g * 16, 16)] = jnp.where(keep, widx, trash)
                    wv = w_v[pl.ds(hwl * DP + g * 16, 16)]
                    for l in range(16):
                        row = u * DP + g * 16 + l
                        w = wv[l]
                        for k in range(5):
                            pay_v[row, pl.ds(k * 16, 16)] = w * fk[k]
            pltpu.sync_copy(pay_v, acc.at[idx_v.at[0]], add=True)
            return carry2

        lax.fori_loop(0, HWT // 2, pair_body, 0)
        plsc.subcore_barrier()

        @pl.when(s < 15)
        def _readout():
            pltpu.sync_copy(acc.at[pl.ds(s * ROWS_PER_SUB, ROWS_PER_SUB)],
                            out_hbm.at[b, c, pl.ds(s * ROWS_PER_SUB, ROWS_PER_SUB)])

        plsc.subcore_barrier()
        return carry

    lax.fori_loop(0, B, batch_body, 0)


_SPLAT_CACHE = {}


def _get_splat_kernel():
    if "k" not in _SPLAT_CACHE:
        _SPLAT_CACHE["k"] = pl.kernel(
            _splat_body,
            out_type=jax.ShapeDtypeStruct((B, 2, ROWS_HALF, OUT_C), jnp.float32),
            mesh=plsc.VectorSubcoreMesh(core_axis_name="c", subcore_axis_name="s",
                                        num_cores=2, num_subcores=16),
            scratch_types=[
                pltpu.VMEM_SHARED((ACC_ROWS, OUT_C), jnp.float32),
                pltpu.VMEM((HWT * OUT_C,), jnp.float32),
                pltpu.VMEM((2 * DP, OUT_C), jnp.float32),
                pltpu.VMEM((HWT * DP,), jnp.float32),
                pltpu.VMEM((HWT * DP,), jnp.float32),
                pltpu.VMEM((HWT * DP,), jnp.float32),
                pltpu.VMEM((HWT * DP,), jnp.float32),
                pltpu.VMEM((1, 2 * DP), jnp.int32),
            ],
            compiler_params=pltpu.CompilerParams(use_tc_tiling_on_sc=False),
        )
    return _SPLAT_CACHE["k"]


def kernel(x_feat, intrins, depth_x, params):
    frustum = _frustum()
    pts = jnp.broadcast_to(frustum[None, None, ..., None], (B, 1, D, FH, FW, 3, 1))
    pts = jnp.concatenate([pts[..., :2, :] * pts[..., 2:3, :], pts[..., 2:3, :]], axis=5)
    rots = jnp.array([[1.0, 0.0, 0.0], [0.0, 0.0, 1.0], [0.0, -1.0, 0.0]], jnp.float32).reshape(1, 1, 3, 3)
    combine = jnp.matmul(rots, jnp.linalg.inv(intrins))
    geom = jnp.matmul(combine.reshape(B, 1, 1, 1, 1, 3, 3), pts)[..., 0]
    gxyz = geom.reshape(B, D, HW, 3).transpose(0, 2, 1, 3)
    gxyz = jnp.pad(gxyz, ((0, 0), (0, 0), (0, DP - D), (0, 0)),
                   constant_values=1e9)
    gxf = gxyz[..., 0].reshape(B * HW * DP)
    gyf = gxyz[..., 1].reshape(B * HW * DP)
    gzf = gxyz[..., 2].reshape(B * HW * DP)
    x = x_feat.reshape(B, IN_C, FH, FW)
    _d = _dtransform(depth_x, params)
    x = jnp.concatenate([_d, x], axis=1)
    x = _depthnet(x, params)
    feat = x.reshape(B, OUT_C, HW).transpose(0, 2, 1).reshape(B * HW * OUT_C)
    avgd = depth_x.reshape(B, 1, FH, 16, FW, 16).mean(axis=(3, 5))
    mean_d = jnp.floor(avgd / 1000.0 * D)
    xg = jnp.broadcast_to(jnp.arange(D, dtype=jnp.float32).reshape(1, D, 1, 1), (B, D, FH, FW))
    sigma = 0.5
    gauss = (1.0 / jnp.sqrt(2.0 * jnp.pi)) * sigma * jnp.exp(-(xg - mean_d) ** 2 / 2.0 * sigma ** 2)
    depth = jax.nn.softmax(gauss, axis=1)
    wdep = depth.reshape(B, D, HW).transpose(0, 2, 1)
    wdep = jnp.pad(wdep, ((0, 0), (0, 0), (0, DP - D))).reshape(B * HW * DP)
    zeros = jnp.zeros((ROWS_HALF, OUT_C), jnp.float32)
    out = _get_splat_kernel()(feat, wdep, gxf, gyf, gzf, zeros)
    out = out.reshape(B, 2, XHALF, NYC, OUT_C).transpose(0, 4, 1, 2, 3)
    return out.reshape(B, OUT_C, NX, NYC)
